# baseline (speedup 1.0000x reference)
import jax
import jax.numpy as jnp
from jax.experimental import pallas as pl
from jax.experimental.pallas import tpu as pltpu

_N = 10000
_D = 128
_BM = 256
_NFULL = _N // _BM
_TAIL = _N - _NFULL * _BM
_NCH = _NFULL + 1
_NBUF = 3


def _body(w_ref, b_ref, x_hbm, adj_hbm, o_hbm,
          xbuf, s_ref, buf0, buf1, buf2, ob0, ob1, ob2,
          in_sems, out_sems, x_sem):
    bufs = (buf0, buf1, buf2)
    obufs = (ob0, ob1, ob2)

    def in_copy(i, slot):
        rows = _BM if i < _NFULL else _TAIL
        src = adj_hbm.at[pl.ds(i * _BM, rows), :]
        dst = bufs[slot].at[pl.ds(0, rows), :]
        return pltpu.make_async_copy(src, dst, in_sems.at[slot])

    def out_copy(i, slot):
        rows = _BM if i < _NFULL else _TAIL
        src = obufs[slot].at[pl.ds(0, rows), :]
        dst = o_hbm.at[pl.ds(i * _BM, rows), :]
        return pltpu.make_async_copy(src, dst, out_sems.at[slot])

    x_copy = pltpu.make_async_copy(x_hbm, xbuf, x_sem)
    x_copy.start()
    for j in range(_NBUF):
        in_copy(j, j).start()

    x_copy.wait()
    s_ref[...] = jnp.dot(xbuf[...], w_ref[...],
                         preferred_element_type=jnp.float32)

    for i in range(_NCH):
        slot = i % _NBUF
        rows = _BM if i < _NFULL else _TAIL
        in_copy(i, slot).wait()
        res = jnp.dot(bufs[slot][0:rows, :], s_ref[...],
                      preferred_element_type=jnp.float32) + b_ref[...]
        if i >= _NBUF:
            out_copy(i - _NBUF, slot).wait()
        obufs[slot][0:rows, :] = res
        out_copy(i, slot).start()
        if i + _NBUF < _NCH:
            in_copy(i + _NBUF, slot).start()

    for i in range(_NCH - _NBUF, _NCH):
        out_copy(i, i % _NBUF).wait()


def kernel(input, adj, W, b):
    n, d_in = input.shape
    d_out = W.shape[1]

    out = pl.pallas_call(
        _body,
        in_specs=[
            pl.BlockSpec((d_in, d_out), lambda: (0, 0)),
            pl.BlockSpec((1, d_out), lambda: (0, 0)),
            pl.BlockSpec(memory_space=pltpu.MemorySpace.HBM),
            pl.BlockSpec(memory_space=pltpu.MemorySpace.HBM),
        ],
        out_specs=pl.BlockSpec(memory_space=pltpu.MemorySpace.HBM),
        out_shape=jax.ShapeDtypeStruct((n, d_out), jnp.float32),
        scratch_shapes=[
            pltpu.VMEM((n, d_in), jnp.float32),
            pltpu.VMEM((n, d_out), jnp.float32),
            pltpu.VMEM((_BM, n), jnp.float32),
            pltpu.VMEM((_BM, n), jnp.float32),
            pltpu.VMEM((_BM, n), jnp.float32),
            pltpu.VMEM((_BM, d_out), jnp.float32),
            pltpu.VMEM((_BM, d_out), jnp.float32),
            pltpu.VMEM((_BM, d_out), jnp.float32),
            pltpu.SemaphoreType.DMA((_NBUF,)),
            pltpu.SemaphoreType.DMA((_NBUF,)),
            pltpu.SemaphoreType.DMA,
        ],
    )(W, b.reshape(1, d_out), input, adj)

    return out
